# baseline (speedup 1.0000x reference)
import functools

import jax
import jax.numpy as jnp
from jax import lax
from jax.experimental import pallas as pl
from jax.experimental.pallas import tpu as pltpu
from jax.experimental.pallas import tpu_sc as plsc

_B = 4
_SEQ = 4096
_D = 1024
_HEADS = 16

_N_IDS = _B * _SEQ
_TC_N = 4096
_SC_N = _N_IDS - _TC_N

_NC = 2
_NS = 16
_NW = _NC * _NS
_PER_W = _SC_N // _NW
_CHUNK = 16
_N_CHUNKS = _PER_W // _CHUNK
_NBUF = 4


def _sc_body(idx_hbm, table_hbm, out_hbm, idx_v, *scratch):
    wid = lax.axis_index("s") * _NC + lax.axis_index("c")
    base = wid * _PER_W
    pltpu.sync_copy(idx_hbm.at[pl.ds(base, _PER_W)], idx_v)

    bufs = scratch[:_NBUF]
    gsems = scratch[_NBUF:2 * _NBUF]
    osems = scratch[2 * _NBUF:]

    def gather(i, b):
        pltpu.make_async_copy(table_hbm.at[idx_v.at[pl.ds(i * _CHUNK, _CHUNK)]],
                              bufs[b], gsems[b]).start()

    def gather_wait(i, b):
        pltpu.make_async_copy(table_hbm.at[idx_v.at[pl.ds(i * _CHUNK, _CHUNK)]],
                              bufs[b], gsems[b]).wait()

    def out_start(i, b):
        pltpu.make_async_copy(bufs[b],
                              out_hbm.at[pl.ds(base + i * _CHUNK, _CHUNK)],
                              osems[b]).start()

    def out_wait(i, b):
        pltpu.make_async_copy(bufs[b],
                              out_hbm.at[pl.ds(base + i * _CHUNK, _CHUNK)],
                              osems[b]).wait()

    for b in range(_NBUF):
        gather(b, b)

    def steady(j, _):
        for b in range(_NBUF):
            i = _NBUF * j + b
            gather_wait(i, b)
            out_start(i, b)
            out_wait(i, b)
            gather(i + _NBUF, b)
        return 0

    lax.fori_loop(0, _N_CHUNKS // _NBUF - 1, steady, 0)

    tail = _N_CHUNKS - _NBUF - (_N_CHUNKS % _NBUF)
    for i in range(tail, _N_CHUNKS - _NBUF):
        b = i % _NBUF
        gather_wait(i, b)
        out_start(i, b)
        out_wait(i, b)
        gather(i + _NBUF, b)
    for i in range(_N_CHUNKS - _NBUF, _N_CHUNKS):
        b = i % _NBUF
        gather_wait(i, b)
        out_start(i, b)
    for i in range(_N_CHUNKS - _NBUF, _N_CHUNKS):
        out_wait(i, i % _NBUF)


@jax.jit
def _sc_gather(ids_flat, table):
    mesh = plsc.VectorSubcoreMesh(core_axis_name="c", subcore_axis_name="s")
    f = functools.partial(
        pl.kernel,
        mesh=mesh,
        out_type=jax.ShapeDtypeStruct((_SC_N, _D), jnp.float32),
        scratch_types=(
            [pltpu.VMEM((_PER_W,), jnp.int32)]
            + [pltpu.VMEM((_CHUNK, _D), jnp.float32)] * _NBUF
            + [pltpu.SemaphoreType.DMA] * (2 * _NBUF)
        ),
    )(_sc_body)
    return f(ids_flat, table)


_TC_R = 256


def _tc_gather_body(ids_ref, table_ref, mask_ref, out_ref, ext_ref, bias_ref,
                    buf0, buf1, gs0, gs1, os0, os1):
    n_rows = out_ref.shape[0]
    nblk = n_rows // _TC_R
    bufs = (buf0, buf1)
    gsems = (gs0, gs1)
    osems = (os0, os1)

    def issue(blk, b):
        base = blk * _TC_R
        for j in range(_TC_R):
            row = ids_ref[base + j]
            pltpu.make_async_copy(table_ref.at[pl.ds(row, 1)],
                                  bufs[b].at[pl.ds(j, 1)], gsems[b]).start()

    def gwait(b):
        pltpu.make_async_copy(table_ref.at[pl.ds(0, _TC_R)], bufs[b],
                              gsems[b]).wait()

    def out_start(blk, b):
        pltpu.make_async_copy(bufs[b], out_ref.at[pl.ds(blk * _TC_R, _TC_R)],
                              osems[b]).start()

    def out_wait(blk, b):
        pltpu.make_async_copy(bufs[b], out_ref.at[pl.ds(blk * _TC_R, _TC_R)],
                              osems[b]).wait()

    issue(0, 0)
    issue(1, 1)

    ext_ref[...] = (1.0 - mask_ref[...]) * -10000.0
    bias_ref[...] = jnp.zeros_like(bias_ref)

    def steady(j, _):
        for b in range(2):
            blk = 2 * j + b
            gwait(b)
            out_start(blk, b)
            out_wait(blk, b)
            issue(blk + 2, b)
        return 0

    lax.fori_loop(0, nblk // 2 - 1, steady, 0)

    for b in range(2):
        gwait(b)
        out_start(nblk - 2 + b, b)
    for b in range(2):
        out_wait(nblk - 2 + b, b)


@jax.jit
def _tc_gather(ids_tc, table, mask):
    return pl.pallas_call(
        _tc_gather_body,
        in_specs=[
            pl.BlockSpec(memory_space=pltpu.SMEM),
            pl.BlockSpec(memory_space=pl.ANY),
            pl.BlockSpec(memory_space=pltpu.VMEM),
        ],
        out_specs=[
            pl.BlockSpec(memory_space=pl.ANY),
            pl.BlockSpec(memory_space=pltpu.VMEM),
            pl.BlockSpec(memory_space=pltpu.VMEM),
        ],
        out_shape=[
            jax.ShapeDtypeStruct((_TC_N, _D), jnp.float32),
            jax.ShapeDtypeStruct((_B, _SEQ), jnp.float32),
            jax.ShapeDtypeStruct((_B * _HEADS, _SEQ), jnp.float32),
        ],
        scratch_shapes=[
            pltpu.VMEM((_TC_R, _D), jnp.float32),
            pltpu.VMEM((_TC_R, _D), jnp.float32),
            pltpu.SemaphoreType.DMA,
            pltpu.SemaphoreType.DMA,
            pltpu.SemaphoreType.DMA,
            pltpu.SemaphoreType.DMA,
        ],
    )(ids_tc, table, mask)


def kernel(encoder_input_ids, encoder_attention_mask, embedding_table):
    ids = encoder_input_ids.astype(jnp.int32).reshape(-1)
    sc_rows = _sc_gather(ids, embedding_table)
    tc_rows, ext, bias = _tc_gather(ids[_SC_N:], embedding_table,
                                    encoder_attention_mask)
    hidden = jnp.concatenate([sc_rows, tc_rows], axis=0).reshape(_B, _SEQ, _D)
    ext = ext.reshape(_B, 1, 1, _SEQ)
    bias = bias.reshape(_B, _HEADS, _SEQ, 1)
    return (hidden, ext, bias)


# speedup vs baseline: 1.7512x; 1.7512x over previous
import functools

import jax
import jax.numpy as jnp
from jax import lax
from jax.experimental import pallas as pl
from jax.experimental.pallas import tpu as pltpu
from jax.experimental.pallas import tpu_sc as plsc

_B = 4
_SEQ = 4096
_D = 1024
_HEADS = 16

_NC = 2
_NS = 16
_NW = _NC * _NS
_N_IDS = _B * _SEQ
_PER_W = _N_IDS // _NW
_CHUNK = 16
_N_CHUNKS = _PER_W // _CHUNK


_NBUF = 6


def _gather_body(idx_hbm, table_hbm, out_hbm, idx_v, *scratch):
    wid = lax.axis_index("s") * _NC + lax.axis_index("c")
    base = wid * _PER_W
    pltpu.sync_copy(idx_hbm.at[pl.ds(base, _PER_W)], idx_v)

    bufs = scratch[:_NBUF]
    gsems = scratch[_NBUF:2 * _NBUF]
    osems = scratch[2 * _NBUF:]

    def gather(i, b):
        pltpu.make_async_copy(table_hbm.at[idx_v.at[pl.ds(i * _CHUNK, _CHUNK)]],
                              bufs[b], gsems[b]).start()

    def gather_wait(i, b):
        pltpu.make_async_copy(table_hbm.at[idx_v.at[pl.ds(i * _CHUNK, _CHUNK)]],
                              bufs[b], gsems[b]).wait()

    def out_start(i, b):
        pltpu.make_async_copy(bufs[b],
                              out_hbm.at[pl.ds(base + i * _CHUNK, _CHUNK)],
                              osems[b]).start()

    def out_wait(i, b):
        pltpu.make_async_copy(bufs[b],
                              out_hbm.at[pl.ds(base + i * _CHUNK, _CHUNK)],
                              osems[b]).wait()

    for b in range(_NBUF):
        gather(b, b)

    def steady(j, _):
        for b in range(_NBUF):
            i = _NBUF * j + b
            gather_wait(i, b)
            out_start(i, b)
            out_wait(i, b)
            gather(i + _NBUF, b)
        return 0

    lax.fori_loop(0, _N_CHUNKS // _NBUF - 1, steady, 0)

    tail = _N_CHUNKS - _NBUF - (_N_CHUNKS % _NBUF)
    for i in range(tail, _N_CHUNKS - _NBUF):
        b = i % _NBUF
        gather_wait(i, b)
        out_start(i, b)
        out_wait(i, b)
        gather(i + _NBUF, b)
    for i in range(_N_CHUNKS - _NBUF, _N_CHUNKS):
        b = i % _NBUF
        gather_wait(i, b)
        out_start(i, b)
    for i in range(_N_CHUNKS - _NBUF, _N_CHUNKS):
        out_wait(i, i % _NBUF)


@jax.jit
def _sc_gather(ids_flat, table):
    mesh = plsc.VectorSubcoreMesh(core_axis_name="c", subcore_axis_name="s")
    f = functools.partial(
        pl.kernel,
        mesh=mesh,
        out_type=jax.ShapeDtypeStruct((_N_IDS, _D), jnp.float32),
        scratch_types=(
            [pltpu.VMEM((_PER_W,), jnp.int32)]
            + [pltpu.VMEM((_CHUNK, _D), jnp.float32)] * _NBUF
            + [pltpu.SemaphoreType.DMA] * (2 * _NBUF)
        ),
    )(_gather_body)
    return f(ids_flat, table)


_TC_R = 256


def _tc_gather_body(ids_ref, table_ref, out_ref, buf0, buf1, gs0, gs1,
                    os0, os1):
    n_rows = out_ref.shape[0]
    nblk = n_rows // _TC_R
    bufs = (buf0, buf1)
    gsems = (gs0, gs1)
    osems = (os0, os1)

    def issue(blk, b):
        base = blk * _TC_R
        for j in range(_TC_R):
            row = ids_ref[base + j]
            pltpu.make_async_copy(table_ref.at[pl.ds(row, 1)],
                                  bufs[b].at[pl.ds(j, 1)], gsems[b]).start()

    def gwait(b):
        pltpu.make_async_copy(table_ref.at[pl.ds(0, _TC_R)], bufs[b],
                              gsems[b]).wait()

    def out_start(blk, b):
        pltpu.make_async_copy(bufs[b], out_ref.at[pl.ds(blk * _TC_R, _TC_R)],
                              osems[b]).start()

    def out_wait(blk, b):
        pltpu.make_async_copy(bufs[b], out_ref.at[pl.ds(blk * _TC_R, _TC_R)],
                              osems[b]).wait()

    issue(0, 0)
    issue(1, 1)

    def steady(j, _):
        for b in range(2):
            blk = 2 * j + b
            gwait(b)
            out_start(blk, b)
            out_wait(blk, b)
            issue(blk + 2, b)
        return 0

    lax.fori_loop(0, nblk // 2 - 1, steady, 0)

    for b in range(2):
        gwait(b)
        out_start(nblk - 2 + b, b)
    for b in range(2):
        out_wait(nblk - 2 + b, b)


@functools.partial(jax.jit, static_argnums=2)
def _tc_gather(ids, table, n_rows):
    return pl.pallas_call(
        _tc_gather_body,
        in_specs=[
            pl.BlockSpec(memory_space=pltpu.SMEM),
            pl.BlockSpec(memory_space=pl.ANY),
        ],
        out_specs=pl.BlockSpec(memory_space=pl.ANY),
        out_shape=jax.ShapeDtypeStruct((n_rows, _D), jnp.float32),
        scratch_shapes=[
            pltpu.VMEM((_TC_R, _D), jnp.float32),
            pltpu.VMEM((_TC_R, _D), jnp.float32),
            pltpu.SemaphoreType.DMA,
            pltpu.SemaphoreType.DMA,
            pltpu.SemaphoreType.DMA,
            pltpu.SemaphoreType.DMA,
        ],
    )(ids, table)


def _mask_body(mask_ref, ext_ref, bias_ref):
    ext_ref[...] = (1.0 - mask_ref[...]) * -10000.0
    bias_ref[...] = jnp.zeros_like(bias_ref)


@jax.jit
def _tc_mask(mask):
    return pl.pallas_call(
        _mask_body,
        out_shape=[
            jax.ShapeDtypeStruct((_B, _SEQ), jnp.float32),
            jax.ShapeDtypeStruct((_B * _HEADS, _SEQ), jnp.float32),
        ],
    )(mask)


def kernel(encoder_input_ids, encoder_attention_mask, embedding_table):
    ids = encoder_input_ids.astype(jnp.int32).reshape(-1)
    hidden = _sc_gather(ids, embedding_table).reshape(_B, _SEQ, _D)
    ext, bias = _tc_mask(encoder_attention_mask)
    ext = ext.reshape(_B, 1, 1, _SEQ)
    bias = bias.reshape(_B, _HEADS, _SEQ, 1)
    return (hidden, ext, bias)
